# baseline (speedup 1.0000x reference)
import functools

import jax
import jax.numpy as jnp
from jax import lax
from jax.experimental import pallas as pl
from jax.experimental.pallas import tpu as pltpu
from jax.experimental.pallas import tpu_sc as plsc

N_PAD = 10240
PAD_NODE = N_PAD - 8
NC = 2
NS = 16
NW = NC * NS
CHUNK = 128
CPT = 80
EPT = CPT * CHUNK
E_PAD = NW * EPT
NBUF = 4
STRIPE = N_PAD // NS


def _sc_mesh():
    return plsc.VectorSubcoreMesh(core_axis_name="c", subcore_axis_name="s")


def _deg_body(dst2d_hbm, zeros1_hbm, out_hbm, dst_v, ones_v, deg_sh, sem):
    cid = lax.axis_index("c")
    sid = lax.axis_index("s")
    gid = cid * NS + sid

    stripe0 = pl.multiple_of(sid * STRIPE, 8)
    pltpu.sync_copy(zeros1_hbm.at[pl.ds(stripe0, STRIPE)],
                    deg_sh.at[pl.ds(stripe0, STRIPE)])
    for j in range(CHUNK // 16):
        ones_v[pl.ds(j * 16, 16)] = jnp.ones((16,), jnp.float32)
    row0 = pl.multiple_of(gid * CPT, 8)
    pltpu.sync_copy(dst2d_hbm.at[pl.ds(row0, CPT)], dst_v)
    plsc.subcore_barrier()

    wave = 8

    def body(w, carry):
        for b in range(wave):
            pltpu.async_copy(ones_v, deg_sh.at[dst_v.at[w * wave + b]], sem,
                             add=True)
        for b in range(wave):
            pltpu.make_async_copy(ones_v, deg_sh.at[dst_v.at[0]], sem).wait()
        return carry

    lax.fori_loop(0, CPT // wave, body, 0)
    plsc.subcore_barrier()
    out0 = pl.multiple_of(cid * N_PAD + sid * STRIPE, 8)
    pltpu.sync_copy(deg_sh.at[pl.ds(stripe0, STRIPE)],
                    out_hbm.at[pl.ds(out0, STRIPE)])


def _make_deg_kernel():
    return pl.kernel(
        _deg_body,
        out_type=jax.ShapeDtypeStruct((NC * N_PAD,), jnp.float32),
        mesh=_sc_mesh(),
        compiler_params=pltpu.CompilerParams(use_tc_tiling_on_sc=False),
        scratch_types=[
            pltpu.VMEM((CPT, CHUNK), jnp.int32),
            pltpu.VMEM((CHUNK,), jnp.float32),
            pltpu.VMEM_SHARED((N_PAD,), jnp.float32),
            pltpu.SemaphoreType.DMA,
        ],
    )


def _agg_body(nbuf, halves, g_hbm, src2d_hbm, dst2d_hbm, zeros2_hbm, out_hbm,
              src_v, dst_v, agg_sh, *bufs):
    rows = bufs[:nbuf]
    gsem = bufs[nbuf:2 * nbuf]
    ssem = bufs[2 * nbuf:3 * nbuf]
    cph = CPT // halves
    cid = lax.axis_index("c")
    sid = lax.axis_index("s")
    gid = cid * NS + sid

    stripe0 = pl.multiple_of(sid * STRIPE, 8)
    pltpu.sync_copy(zeros2_hbm.at[pl.ds(stripe0, STRIPE)],
                    agg_sh.at[pl.ds(stripe0, STRIPE)])
    plsc.subcore_barrier()

    for h in range(halves):
        row0 = pl.multiple_of(gid * CPT + h * cph, 8)
        pltpu.sync_copy(src2d_hbm.at[pl.ds(row0, cph)], src_v)
        pltpu.sync_copy(dst2d_hbm.at[pl.ds(row0, cph)], dst_v)

        for b in range(nbuf):
            pltpu.async_copy(g_hbm.at[src_v.at[b]], rows[b], gsem[b])

        def body(j0, carry):
            for b in range(nbuf):
                j = j0 * nbuf + b
                pltpu.make_async_copy(g_hbm.at[src_v.at[0]], rows[b],
                                      gsem[b]).wait()
                pltpu.async_copy(rows[b], agg_sh.at[dst_v.at[j]], ssem[b],
                                 add=True)
                pltpu.make_async_copy(rows[b], agg_sh.at[dst_v.at[0]],
                                      ssem[b]).wait()
                jn = j + nbuf

                @pl.when(jn < cph)
                def _():
                    pltpu.async_copy(g_hbm.at[src_v.at[jn]], rows[b], gsem[b])

            return carry

        lax.fori_loop(0, cph // nbuf, body, 0)

    plsc.subcore_barrier()
    out0 = pl.multiple_of(cid * N_PAD + sid * STRIPE, 8)
    pltpu.sync_copy(agg_sh.at[pl.ds(stripe0, STRIPE)],
                    out_hbm.at[pl.ds(out0, STRIPE)])


def _make_agg_kernel(d_model, nbuf, halves):
    return pl.kernel(
        functools.partial(_agg_body, nbuf, halves),
        out_type=jax.ShapeDtypeStruct((NC * N_PAD, d_model), jnp.float32),
        mesh=_sc_mesh(),
        compiler_params=pltpu.CompilerParams(use_tc_tiling_on_sc=False),
        scratch_types=[
            pltpu.VMEM((CPT // halves, CHUNK), jnp.int32),
            pltpu.VMEM((CPT // halves, CHUNK), jnp.int32),
            pltpu.VMEM_SHARED((N_PAD, d_model), jnp.float32),
        ] + [pltpu.VMEM((CHUNK, d_model), jnp.float32)] * nbuf
          + [pltpu.SemaphoreType.DMA] * (2 * nbuf),
    )


def _lin1_body(x_ref, w_ref, deg_ref, g_ref, dis_ref):
    deg = deg_ref[0, :] + deg_ref[1, :] + 1.0
    dis = jnp.where(deg > 0, lax.rsqrt(deg), 0.0)
    h = jnp.dot(x_ref[...], w_ref[...], preferred_element_type=jnp.float32)
    g_ref[...] = h * dis[:, None]
    dis_ref[...] = dis[:, None]


def _lin2_body(agg_ref, g1_ref, dis_ref, w_ref, b_ref, g2_ref):
    dis = dis_ref[...]
    agg = agg_ref[0] + agg_ref[1] + g1_ref[...]
    z = jnp.maximum(agg * dis + b_ref[...], 0.0)
    h2 = jnp.dot(z, w_ref[...], preferred_element_type=jnp.float32)
    g2_ref[...] = h2 * dis


def _out_body(agg_ref, g2_ref, dis_ref, b_ref, o_ref):
    agg = agg_ref[0] + agg_ref[1] + g2_ref[...]
    o_ref[...] = agg * dis_ref[...] + b_ref[...]


def _lin1(xp, w1, deg2):
    d_in, d_hid = w1.shape
    grid = (N_PAD // STRIPE,)
    return pl.pallas_call(
        _lin1_body,
        grid=grid,
        in_specs=[
            pl.BlockSpec((STRIPE, d_in), lambda i: (i, 0)),
            pl.BlockSpec((d_in, d_hid), lambda i: (0, 0)),
            pl.BlockSpec((NC, STRIPE), lambda i: (0, i)),
        ],
        out_specs=[
            pl.BlockSpec((STRIPE, d_hid), lambda i: (i, 0)),
            pl.BlockSpec((STRIPE, 1), lambda i: (i, 0)),
        ],
        out_shape=[
            jax.ShapeDtypeStruct((N_PAD, d_hid), jnp.float32),
            jax.ShapeDtypeStruct((N_PAD, 1), jnp.float32),
        ],
    )(xp, w1, deg2)


def _lin2(agg1, g1, dis, w2, b1):
    d_hid, d_out = w2.shape
    grid = (N_PAD // STRIPE,)
    return pl.pallas_call(
        _lin2_body,
        grid=grid,
        in_specs=[
            pl.BlockSpec((NC, STRIPE, d_hid), lambda i: (0, i, 0)),
            pl.BlockSpec((STRIPE, d_hid), lambda i: (i, 0)),
            pl.BlockSpec((STRIPE, 1), lambda i: (i, 0)),
            pl.BlockSpec((d_hid, d_out), lambda i: (0, 0)),
            pl.BlockSpec((1, d_hid), lambda i: (0, 0)),
        ],
        out_specs=pl.BlockSpec((STRIPE, d_out), lambda i: (i, 0)),
        out_shape=jax.ShapeDtypeStruct((N_PAD, d_out), jnp.float32),
    )(agg1, g1, dis, w2, b1)


def _outk(agg2, g2, dis, b2):
    d_out = g2.shape[1]
    grid = (N_PAD // STRIPE,)
    return pl.pallas_call(
        _out_body,
        grid=grid,
        in_specs=[
            pl.BlockSpec((NC, STRIPE, d_out), lambda i: (0, i, 0)),
            pl.BlockSpec((STRIPE, d_out), lambda i: (i, 0)),
            pl.BlockSpec((STRIPE, 1), lambda i: (i, 0)),
            pl.BlockSpec((1, d_out), lambda i: (0, 0)),
        ],
        out_specs=pl.BlockSpec((STRIPE, d_out), lambda i: (i, 0)),
        out_shape=jax.ShapeDtypeStruct((N_PAD, d_out), jnp.float32),
    )(agg2, g2, dis, b2)


@jax.jit
def kernel(x, edge_index, W1, b1, W2, b2):
    n, d_in = x.shape
    d_hid = W1.shape[1]
    d_out = W2.shape[1]
    n_edges = edge_index.shape[1]
    pad = jnp.full((E_PAD - n_edges,), PAD_NODE, jnp.int32)
    src2d = jnp.concatenate(
        [edge_index[0].astype(jnp.int32), pad]).reshape(E_PAD // CHUNK, CHUNK)
    dst2d = jnp.concatenate(
        [edge_index[1].astype(jnp.int32), pad]).reshape(E_PAD // CHUNK, CHUNK)

    xp = jnp.zeros((N_PAD, d_in), jnp.float32).at[:n].set(x)
    zeros1 = jnp.zeros((N_PAD,), jnp.float32)
    zeros_h = jnp.zeros((N_PAD, d_hid), jnp.float32)
    zeros_o = jnp.zeros((N_PAD, d_out), jnp.float32)

    deg2 = _make_deg_kernel()(dst2d, zeros1).reshape(NC, N_PAD)
    g1, dis = _lin1(xp, W1, deg2)
    agg1 = _make_agg_kernel(d_hid, 2, 2)(g1, src2d, dst2d, zeros_h)
    agg1 = agg1.reshape(NC, N_PAD, d_hid)
    g2 = _lin2(agg1, g1, dis, W2, b1.reshape(1, d_hid))
    agg2 = _make_agg_kernel(d_out, 4, 1)(g2, src2d, dst2d, zeros_o)
    agg2 = agg2.reshape(NC, N_PAD, d_out)
    out = _outk(agg2, g2, dis, b2.reshape(1, d_out))
    return out[:n]


# speedup vs baseline: 2.9371x; 2.9371x over previous
import functools

import jax
import jax.numpy as jnp
from jax import lax
from jax.experimental import pallas as pl
from jax.experimental.pallas import tpu as pltpu
from jax.experimental.pallas import tpu_sc as plsc

N_PAD = 10240
PAD_NODE = N_PAD - 8
NC = 2
NS = 16
NW = NC * NS
CHUNK = 128
CPT = 80
EPT = CPT * CHUNK
E_PAD = NW * EPT
NBUF = 4
STRIPE = N_PAD // NS


def _sc_mesh():
    return plsc.VectorSubcoreMesh(core_axis_name="c", subcore_axis_name="s")


def _deg_body(dst2d_hbm, zeros1_hbm, out_hbm, dst_v, ones_v, deg_sh, sem):
    cid = lax.axis_index("c")
    sid = lax.axis_index("s")
    gid = cid * NS + sid

    stripe0 = pl.multiple_of(sid * STRIPE, 8)
    pltpu.sync_copy(zeros1_hbm.at[pl.ds(stripe0, STRIPE)],
                    deg_sh.at[pl.ds(stripe0, STRIPE)])
    for j in range(CHUNK // 16):
        ones_v[pl.ds(j * 16, 16)] = jnp.ones((16,), jnp.float32)
    row0 = pl.multiple_of(gid * CPT, 8)
    pltpu.sync_copy(dst2d_hbm.at[pl.ds(row0, CPT)], dst_v)
    plsc.subcore_barrier()

    wave = 8

    def body(w, carry):
        for b in range(wave):
            pltpu.async_copy(ones_v, deg_sh.at[dst_v.at[w * wave + b]], sem,
                             add=True)
        for b in range(wave):
            pltpu.make_async_copy(ones_v, deg_sh.at[dst_v.at[0]], sem).wait()
        return carry

    lax.fori_loop(0, CPT // wave, body, 0)
    plsc.subcore_barrier()
    out0 = pl.multiple_of(cid * N_PAD + sid * STRIPE, 8)
    pltpu.sync_copy(deg_sh.at[pl.ds(stripe0, STRIPE)],
                    out_hbm.at[pl.ds(out0, STRIPE)])


def _make_deg_kernel():
    return pl.kernel(
        _deg_body,
        out_type=jax.ShapeDtypeStruct((NC * N_PAD,), jnp.float32),
        mesh=_sc_mesh(),
        compiler_params=pltpu.CompilerParams(use_tc_tiling_on_sc=False),
        scratch_types=[
            pltpu.VMEM((CPT, CHUNK), jnp.int32),
            pltpu.VMEM((CHUNK,), jnp.float32),
            pltpu.VMEM_SHARED((N_PAD,), jnp.float32),
            pltpu.SemaphoreType.DMA,
        ],
    )


def _agg_body(nbuf, halves, g_hbm, src2d_hbm, dst2d_hbm, zeros2_hbm, out_hbm,
              src_v, dst_v, agg_sh, *bufs):
    rows = bufs[:nbuf]
    gsem = bufs[nbuf:2 * nbuf]
    ssem = bufs[2 * nbuf:3 * nbuf]
    cph = CPT // halves
    cid = lax.axis_index("c")
    sid = lax.axis_index("s")
    gid = cid * NS + sid

    stripe0 = pl.multiple_of(sid * STRIPE, 8)
    pltpu.sync_copy(zeros2_hbm.at[pl.ds(stripe0, STRIPE)],
                    agg_sh.at[pl.ds(stripe0, STRIPE)])
    plsc.subcore_barrier()

    for h in range(halves):
        row0 = pl.multiple_of(gid * CPT + h * cph, 8)
        pltpu.sync_copy(src2d_hbm.at[pl.ds(row0, cph)], src_v)
        pltpu.sync_copy(dst2d_hbm.at[pl.ds(row0, cph)], dst_v)

        for b in range(nbuf):
            pltpu.async_copy(g_hbm.at[src_v.at[b]], rows[b], gsem[b])

        def body(j0, carry):
            for b in range(nbuf):
                j = j0 * nbuf + b
                pltpu.make_async_copy(g_hbm.at[src_v.at[0]], rows[b],
                                      gsem[b]).wait()
                pltpu.async_copy(rows[b], agg_sh.at[dst_v.at[j]], ssem[b],
                                 add=True)
                pltpu.make_async_copy(rows[b], agg_sh.at[dst_v.at[0]],
                                      ssem[b]).wait()
                jn = j + nbuf

                @pl.when(jn < cph)
                def _():
                    pltpu.async_copy(g_hbm.at[src_v.at[jn]], rows[b], gsem[b])

            return carry

        lax.fori_loop(0, cph // nbuf, body, 0)

    plsc.subcore_barrier()
    out0 = pl.multiple_of(cid * N_PAD + sid * STRIPE, 8)
    pltpu.sync_copy(agg_sh.at[pl.ds(stripe0, STRIPE)],
                    out_hbm.at[pl.ds(out0, STRIPE)])


def _make_agg_kernel(d_model, nbuf, halves):
    return pl.kernel(
        functools.partial(_agg_body, nbuf, halves),
        out_type=jax.ShapeDtypeStruct((NC * N_PAD, d_model), jnp.float32),
        mesh=_sc_mesh(),
        compiler_params=pltpu.CompilerParams(use_tc_tiling_on_sc=False),
        scratch_types=[
            pltpu.VMEM((CPT // halves, CHUNK), jnp.int32),
            pltpu.VMEM((CPT // halves, CHUNK), jnp.int32),
            pltpu.VMEM_SHARED((N_PAD, d_model), jnp.float32),
        ] + [pltpu.VMEM((CHUNK, d_model), jnp.float32)] * nbuf
          + [pltpu.SemaphoreType.DMA] * (2 * nbuf),
    )


def _lin1_body(x_ref, w_ref, deg_ref, g_ref, dis_ref):
    deg = deg_ref[0, :] + deg_ref[1, :] + 1.0
    dis = jnp.where(deg > 0, lax.rsqrt(deg), 0.0)
    h = jnp.dot(x_ref[...], w_ref[...], preferred_element_type=jnp.float32)
    g_ref[...] = h * dis[:, None]
    dis_ref[...] = dis[:, None]


def _lin2_body(agg_ref, g1_ref, dis_ref, w_ref, b_ref, g2_ref):
    dis = dis_ref[...]
    agg = agg_ref[0] + agg_ref[1] + g1_ref[...]
    z = jnp.maximum(agg * dis + b_ref[...], 0.0)
    h2 = jnp.dot(z, w_ref[...], preferred_element_type=jnp.float32)
    g2_ref[...] = h2 * dis


def _out_body(agg_ref, g2_ref, dis_ref, b_ref, o_ref):
    agg = agg_ref[0] + agg_ref[1] + g2_ref[...]
    o_ref[...] = agg * dis_ref[...] + b_ref[...]


def _lin1(xp, w1, deg2):
    d_in, d_hid = w1.shape
    grid = (N_PAD // STRIPE,)
    return pl.pallas_call(
        _lin1_body,
        grid=grid,
        in_specs=[
            pl.BlockSpec((STRIPE, d_in), lambda i: (i, 0)),
            pl.BlockSpec((d_in, d_hid), lambda i: (0, 0)),
            pl.BlockSpec((NC, STRIPE), lambda i: (0, i)),
        ],
        out_specs=[
            pl.BlockSpec((STRIPE, d_hid), lambda i: (i, 0)),
            pl.BlockSpec((STRIPE, 1), lambda i: (i, 0)),
        ],
        out_shape=[
            jax.ShapeDtypeStruct((N_PAD, d_hid), jnp.float32),
            jax.ShapeDtypeStruct((N_PAD, 1), jnp.float32),
        ],
    )(xp, w1, deg2)


def _lin2(agg1, g1, dis, w2, b1):
    d_hid, d_out = w2.shape
    grid = (N_PAD // STRIPE,)
    return pl.pallas_call(
        _lin2_body,
        grid=grid,
        in_specs=[
            pl.BlockSpec((NC, STRIPE, d_hid), lambda i: (0, i, 0)),
            pl.BlockSpec((STRIPE, d_hid), lambda i: (i, 0)),
            pl.BlockSpec((STRIPE, 1), lambda i: (i, 0)),
            pl.BlockSpec((d_hid, d_out), lambda i: (0, 0)),
            pl.BlockSpec((1, d_hid), lambda i: (0, 0)),
        ],
        out_specs=pl.BlockSpec((STRIPE, d_out), lambda i: (i, 0)),
        out_shape=jax.ShapeDtypeStruct((N_PAD, d_out), jnp.float32),
    )(agg1, g1, dis, w2, b1)


def _outk(agg2, g2, dis, b2):
    d_out = g2.shape[1]
    grid = (N_PAD // STRIPE,)
    return pl.pallas_call(
        _out_body,
        grid=grid,
        in_specs=[
            pl.BlockSpec((NC, STRIPE, d_out), lambda i: (0, i, 0)),
            pl.BlockSpec((STRIPE, d_out), lambda i: (i, 0)),
            pl.BlockSpec((STRIPE, 1), lambda i: (i, 0)),
            pl.BlockSpec((1, d_out), lambda i: (0, 0)),
        ],
        out_specs=pl.BlockSpec((STRIPE, d_out), lambda i: (i, 0)),
        out_shape=jax.ShapeDtypeStruct((N_PAD, d_out), jnp.float32),
    )(agg2, g2, dis, b2)


@jax.jit
def kernel(x, edge_index, W1, b1, W2, b2):
    n, d_in = x.shape
    d_hid = W1.shape[1]
    d_out = W2.shape[1]
    n_edges = edge_index.shape[1]
    pad = n + jnp.arange(E_PAD - n_edges, dtype=jnp.int32) % (N_PAD - n)
    src2d = jnp.concatenate(
        [edge_index[0].astype(jnp.int32), pad]).reshape(E_PAD // CHUNK, CHUNK)
    dst2d = jnp.concatenate(
        [edge_index[1].astype(jnp.int32), pad]).reshape(E_PAD // CHUNK, CHUNK)

    xp = jnp.zeros((N_PAD, d_in), jnp.float32).at[:n].set(x)
    zeros1 = jnp.zeros((N_PAD,), jnp.float32)
    zeros_h = jnp.zeros((N_PAD, d_hid), jnp.float32)
    zeros_o = jnp.zeros((N_PAD, d_out), jnp.float32)

    deg2 = _make_deg_kernel()(dst2d, zeros1).reshape(NC, N_PAD)
    g1, dis = _lin1(xp, W1, deg2)
    agg1 = _make_agg_kernel(d_hid, 2, 2)(g1, src2d, dst2d, zeros_h)
    agg1 = agg1.reshape(NC, N_PAD, d_hid)
    g2 = _lin2(agg1, g1, dis, W2, b1.reshape(1, d_hid))
    agg2 = _make_agg_kernel(d_out, 4, 1)(g2, src2d, dst2d, zeros_o)
    agg2 = agg2.reshape(NC, N_PAD, d_out)
    out = _outk(agg2, g2, dis, b2.reshape(1, d_out))
    return out[:n]


# speedup vs baseline: 3.2376x; 1.1023x over previous
import functools

import jax
import jax.numpy as jnp
from jax import lax
from jax.experimental import pallas as pl
from jax.experimental.pallas import tpu as pltpu
from jax.experimental.pallas import tpu_sc as plsc

N_PAD = 10240
PAD_NODE = N_PAD - 8
NC = 2
NS = 16
NW = NC * NS
CHUNK = 128
CPT = 80
EPT = CPT * CHUNK
E_PAD = NW * EPT
NBUF = 4
STRIPE = N_PAD // NS


def _sc_mesh():
    return plsc.VectorSubcoreMesh(core_axis_name="c", subcore_axis_name="s")


def _deg_body(dst2d_hbm, zeros1_hbm, out_hbm, dst_v, ones_v, deg_sh, sem):
    cid = lax.axis_index("c")
    sid = lax.axis_index("s")
    gid = cid * NS + sid

    stripe0 = pl.multiple_of(sid * STRIPE, 8)
    pltpu.sync_copy(zeros1_hbm.at[pl.ds(stripe0, STRIPE)],
                    deg_sh.at[pl.ds(stripe0, STRIPE)])
    for j in range(CHUNK // 16):
        ones_v[pl.ds(j * 16, 16)] = jnp.ones((16,), jnp.float32)
    row0 = pl.multiple_of(gid * CPT, 8)
    pltpu.sync_copy(dst2d_hbm.at[pl.ds(row0, CPT)], dst_v)
    plsc.subcore_barrier()

    wave = 8

    def body(w, carry):
        for b in range(wave):
            pltpu.async_copy(ones_v, deg_sh.at[dst_v.at[w * wave + b]], sem,
                             add=True)
        for b in range(wave):
            pltpu.make_async_copy(ones_v, deg_sh.at[dst_v.at[0]], sem).wait()
        return carry

    lax.fori_loop(0, CPT // wave, body, 0)
    plsc.subcore_barrier()
    out0 = pl.multiple_of(cid * N_PAD + sid * STRIPE, 8)
    pltpu.sync_copy(deg_sh.at[pl.ds(stripe0, STRIPE)],
                    out_hbm.at[pl.ds(out0, STRIPE)])


def _make_deg_kernel():
    return pl.kernel(
        _deg_body,
        out_type=jax.ShapeDtypeStruct((NC * N_PAD,), jnp.float32),
        mesh=_sc_mesh(),
        compiler_params=pltpu.CompilerParams(use_tc_tiling_on_sc=False),
        scratch_types=[
            pltpu.VMEM((CPT, CHUNK), jnp.int32),
            pltpu.VMEM((CHUNK,), jnp.float32),
            pltpu.VMEM_SHARED((N_PAD,), jnp.float32),
            pltpu.SemaphoreType.DMA,
        ],
    )


def _agg_body(chunk, nbuf, halves, g_hbm, src2d_hbm, dst2d_hbm, zeros2_hbm,
              out_hbm, src_v, dst_v, agg_sh, *bufs):
    rows = bufs[:nbuf]
    gsem = bufs[nbuf:2 * nbuf]
    ssem = bufs[2 * nbuf:3 * nbuf]
    cpt = EPT // chunk
    cph = cpt // halves
    cid = lax.axis_index("c")
    sid = lax.axis_index("s")
    gid = cid * NS + sid

    stripe0 = pl.multiple_of(sid * STRIPE, 8)
    pltpu.sync_copy(zeros2_hbm.at[pl.ds(stripe0, STRIPE)],
                    agg_sh.at[pl.ds(stripe0, STRIPE)])
    plsc.subcore_barrier()

    for h in range(halves):
        row0 = pl.multiple_of(gid * cpt + h * cph, 8)
        pltpu.sync_copy(src2d_hbm.at[pl.ds(row0, cph)], src_v)
        pltpu.sync_copy(dst2d_hbm.at[pl.ds(row0, cph)], dst_v)

        for b in range(nbuf):
            pltpu.async_copy(g_hbm.at[src_v.at[b]], rows[b], gsem[b])

        def body(j0, carry):
            for b in range(nbuf):
                j = j0 * nbuf + b
                bp = (b - 1) % nbuf

                def drain_and_regather():
                    pltpu.make_async_copy(rows[bp], agg_sh.at[dst_v.at[0]],
                                          ssem[bp]).wait()
                    jn = j - 1 + nbuf

                    @pl.when(jn < cph)
                    def _():
                        pltpu.async_copy(g_hbm.at[src_v.at[jn]], rows[bp],
                                         gsem[bp])

                if b == 0:
                    pl.when(j0 >= 1)(drain_and_regather)
                else:
                    drain_and_regather()

                pltpu.make_async_copy(g_hbm.at[src_v.at[0]], rows[b],
                                      gsem[b]).wait()
                pltpu.async_copy(rows[b], agg_sh.at[dst_v.at[j]], ssem[b],
                                 add=True)

            return carry

        lax.fori_loop(0, cph // nbuf, body, 0)
        pltpu.make_async_copy(rows[nbuf - 1], agg_sh.at[dst_v.at[0]],
                              ssem[nbuf - 1]).wait()

    plsc.subcore_barrier()
    out0 = pl.multiple_of(cid * N_PAD + sid * STRIPE, 8)
    pltpu.sync_copy(agg_sh.at[pl.ds(stripe0, STRIPE)],
                    out_hbm.at[pl.ds(out0, STRIPE)])


def _make_agg_kernel(d_model, chunk, nbuf, halves):
    cph = (EPT // chunk) // halves
    return pl.kernel(
        functools.partial(_agg_body, chunk, nbuf, halves),
        out_type=jax.ShapeDtypeStruct((NC * N_PAD, d_model), jnp.float32),
        mesh=_sc_mesh(),
        compiler_params=pltpu.CompilerParams(use_tc_tiling_on_sc=False),
        scratch_types=[
            pltpu.VMEM((cph, chunk), jnp.int32),
            pltpu.VMEM((cph, chunk), jnp.int32),
            pltpu.VMEM_SHARED((N_PAD, d_model), jnp.float32),
        ] + [pltpu.VMEM((chunk, d_model), jnp.float32)] * nbuf
          + [pltpu.SemaphoreType.DMA] * (2 * nbuf),
    )


def _lin1_body(n, x_ref, w_ref, deg_ref, g_ref, dis_ref):
    deg = deg_ref[0, :] + deg_ref[1, :] + 1.0
    dis = jnp.where(deg > 0, lax.rsqrt(deg), 0.0)
    h = jnp.dot(x_ref[...], w_ref[...], preferred_element_type=jnp.float32)
    g_ref[pl.ds(0, n), :] = h * dis[:n, None]
    g_ref[pl.ds(n, N_PAD - n), :] = jnp.zeros((N_PAD - n, h.shape[1]),
                                              jnp.float32)
    dis_ref[...] = dis[:, None]


def _lin2_body(agg_ref, g1_ref, dis_ref, w_ref, b_ref, g2_ref):
    dis = dis_ref[...]
    agg = agg_ref[0] + agg_ref[1] + g1_ref[...]
    z = jnp.maximum(agg * dis + b_ref[...], 0.0)
    h2 = jnp.dot(z, w_ref[...], preferred_element_type=jnp.float32)
    g2_ref[...] = h2 * dis


def _out_body(n, agg_ref, g2_ref, dis_ref, b_ref, o_ref):
    agg = agg_ref[0] + agg_ref[1] + g2_ref[...]
    o_ref[...] = (agg * dis_ref[...] + b_ref[...])[:n]


def _lin1(x, w1, deg2):
    n, d_in = x.shape
    d_hid = w1.shape[1]
    return pl.pallas_call(
        functools.partial(_lin1_body, n),
        out_shape=[
            jax.ShapeDtypeStruct((N_PAD, d_hid), jnp.float32),
            jax.ShapeDtypeStruct((N_PAD, 1), jnp.float32),
        ],
    )(x, w1, deg2)


def _lin2(agg1, g1, dis, w2, b1):
    d_hid, d_out = w2.shape
    grid = (N_PAD // STRIPE,)
    return pl.pallas_call(
        _lin2_body,
        grid=grid,
        in_specs=[
            pl.BlockSpec((NC, STRIPE, d_hid), lambda i: (0, i, 0)),
            pl.BlockSpec((STRIPE, d_hid), lambda i: (i, 0)),
            pl.BlockSpec((STRIPE, 1), lambda i: (i, 0)),
            pl.BlockSpec((d_hid, d_out), lambda i: (0, 0)),
            pl.BlockSpec((1, d_hid), lambda i: (0, 0)),
        ],
        out_specs=pl.BlockSpec((STRIPE, d_out), lambda i: (i, 0)),
        out_shape=jax.ShapeDtypeStruct((N_PAD, d_out), jnp.float32),
    )(agg1, g1, dis, w2, b1)


def _outk(n, agg2, g2, dis, b2):
    d_out = g2.shape[1]
    return pl.pallas_call(
        functools.partial(_out_body, n),
        out_shape=jax.ShapeDtypeStruct((n, d_out), jnp.float32),
    )(agg2, g2, dis, b2)


@jax.jit
def kernel(x, edge_index, W1, b1, W2, b2):
    n, d_in = x.shape
    d_hid = W1.shape[1]
    d_out = W2.shape[1]
    n_edges = edge_index.shape[1]
    pad = n + jnp.arange(E_PAD - n_edges, dtype=jnp.int32) % (N_PAD - n)
    src1 = jnp.concatenate([edge_index[0].astype(jnp.int32), pad])
    dst1 = jnp.concatenate([edge_index[1].astype(jnp.int32), pad])

    zeros1 = jnp.zeros((N_PAD,), jnp.float32)
    zeros_h = jnp.zeros((N_PAD, d_hid), jnp.float32)
    zeros_o = jnp.zeros((N_PAD, d_out), jnp.float32)

    deg2 = _make_deg_kernel()(
        dst1.reshape(E_PAD // CHUNK, CHUNK), zeros1).reshape(NC, N_PAD)
    g1, dis = _lin1(x, W1, deg2)
    c1 = 64
    agg1 = _make_agg_kernel(d_hid, c1, 4, 2)(
        g1, src1.reshape(E_PAD // c1, c1), dst1.reshape(E_PAD // c1, c1),
        zeros_h)
    agg1 = agg1.reshape(NC, N_PAD, d_hid)
    g2 = _lin2(agg1, g1, dis, W2, b1.reshape(1, d_hid))
    c2 = 128
    agg2 = _make_agg_kernel(d_out, c2, 4, 1)(
        g2, src1.reshape(E_PAD // c2, c2), dst1.reshape(E_PAD // c2, c2),
        zeros_o)
    agg2 = agg2.reshape(NC, N_PAD, d_out)
    return _outk(n, agg2, g2, dis, b2.reshape(1, d_out))


# speedup vs baseline: 3.3334x; 1.0296x over previous
import functools

import jax
import jax.numpy as jnp
from jax import lax
from jax.experimental import pallas as pl
from jax.experimental.pallas import tpu as pltpu
from jax.experimental.pallas import tpu_sc as plsc

N_PAD = 10240
PAD_NODE = N_PAD - 8
NC = 2
NS = 16
NW = NC * NS
CHUNK = 128
CPT = 80
EPT = CPT * CHUNK
E_PAD = NW * EPT
NBUF = 4
STRIPE = N_PAD // NS


def _sc_mesh():
    return plsc.VectorSubcoreMesh(core_axis_name="c", subcore_axis_name="s")


def _deg_body(dst2d_hbm, zeros1_hbm, out_hbm, dst_v, ones_v, deg_sh, sem):
    cid = lax.axis_index("c")
    sid = lax.axis_index("s")
    gid = cid * NS + sid

    stripe0 = pl.multiple_of(sid * STRIPE, 8)
    pltpu.sync_copy(zeros1_hbm.at[pl.ds(stripe0, STRIPE)],
                    deg_sh.at[pl.ds(stripe0, STRIPE)])
    for j in range(CHUNK // 16):
        ones_v[pl.ds(j * 16, 16)] = jnp.ones((16,), jnp.float32)
    row0 = pl.multiple_of(gid * CPT, 8)
    pltpu.sync_copy(dst2d_hbm.at[pl.ds(row0, CPT)], dst_v)
    plsc.subcore_barrier()

    wave = 8

    def body(w, carry):
        for b in range(wave):
            pltpu.async_copy(ones_v, deg_sh.at[dst_v.at[w * wave + b]], sem,
                             add=True)
        for b in range(wave):
            pltpu.make_async_copy(ones_v, deg_sh.at[dst_v.at[0]], sem).wait()
        return carry

    lax.fori_loop(0, CPT // wave, body, 0)
    plsc.subcore_barrier()
    pltpu.sync_copy(deg_sh.at[pl.ds(stripe0, STRIPE)],
                    out_hbm.at[cid, pl.ds(stripe0, STRIPE)])


def _make_deg_kernel():
    return pl.kernel(
        _deg_body,
        out_type=jax.ShapeDtypeStruct((NC, N_PAD), jnp.float32),
        mesh=_sc_mesh(),
        compiler_params=pltpu.CompilerParams(use_tc_tiling_on_sc=False),
        scratch_types=[
            pltpu.VMEM((CPT, CHUNK), jnp.int32),
            pltpu.VMEM((CHUNK,), jnp.float32),
            pltpu.VMEM_SHARED((N_PAD,), jnp.float32),
            pltpu.SemaphoreType.DMA,
        ],
    )


def _agg_body(chunk, nbuf, halves, g_hbm, src2d_hbm, dst2d_hbm, zeros2_hbm,
              out0_hbm, out1_hbm, src_v, dst_v, agg_sh, *bufs):
    rows = bufs[:nbuf]
    gsem = bufs[nbuf:2 * nbuf]
    ssem = bufs[2 * nbuf:3 * nbuf]
    cpt = EPT // chunk
    cph = cpt // halves
    cid = lax.axis_index("c")
    sid = lax.axis_index("s")
    gid = cid * NS + sid

    stripe0 = pl.multiple_of(sid * STRIPE, 8)
    pltpu.sync_copy(zeros2_hbm.at[pl.ds(stripe0, STRIPE)],
                    agg_sh.at[pl.ds(stripe0, STRIPE)])
    plsc.subcore_barrier()

    for h in range(halves):
        row0 = pl.multiple_of(gid * cpt + h * cph, 8)
        pltpu.sync_copy(src2d_hbm.at[pl.ds(row0, cph)], src_v)
        pltpu.sync_copy(dst2d_hbm.at[pl.ds(row0, cph)], dst_v)

        for b in range(nbuf):
            pltpu.async_copy(g_hbm.at[src_v.at[b]], rows[b], gsem[b])

        def body(j0, carry):
            for b in range(nbuf):
                j = j0 * nbuf + b
                bp = (b - 1) % nbuf

                def drain_and_regather():
                    pltpu.make_async_copy(rows[bp], agg_sh.at[dst_v.at[0]],
                                          ssem[bp]).wait()
                    jn = j - 1 + nbuf

                    @pl.when(jn < cph)
                    def _():
                        pltpu.async_copy(g_hbm.at[src_v.at[jn]], rows[bp],
                                         gsem[bp])

                if b == 0:
                    pl.when(j0 >= 1)(drain_and_regather)
                else:
                    drain_and_regather()

                pltpu.make_async_copy(g_hbm.at[src_v.at[0]], rows[b],
                                      gsem[b]).wait()
                pltpu.async_copy(rows[b], agg_sh.at[dst_v.at[j]], ssem[b],
                                 add=True)

            return carry

        lax.fori_loop(0, cph // nbuf, body, 0)
        pltpu.make_async_copy(rows[nbuf - 1], agg_sh.at[dst_v.at[0]],
                              ssem[nbuf - 1]).wait()

    plsc.subcore_barrier()

    @pl.when(cid == 0)
    def _():
        pltpu.sync_copy(agg_sh.at[pl.ds(stripe0, STRIPE)],
                        out0_hbm.at[pl.ds(stripe0, STRIPE)])

    @pl.when(cid == 1)
    def _():
        pltpu.sync_copy(agg_sh.at[pl.ds(stripe0, STRIPE)],
                        out1_hbm.at[pl.ds(stripe0, STRIPE)])


def _make_agg_kernel(d_model, chunk, nbuf, halves):
    cph = (EPT // chunk) // halves
    return pl.kernel(
        functools.partial(_agg_body, chunk, nbuf, halves),
        out_type=[jax.ShapeDtypeStruct((N_PAD, d_model), jnp.float32),
                  jax.ShapeDtypeStruct((N_PAD, d_model), jnp.float32)],
        mesh=_sc_mesh(),
        compiler_params=pltpu.CompilerParams(use_tc_tiling_on_sc=False),
        scratch_types=[
            pltpu.VMEM((cph, chunk), jnp.int32),
            pltpu.VMEM((cph, chunk), jnp.int32),
            pltpu.VMEM_SHARED((N_PAD, d_model), jnp.float32),
        ] + [pltpu.VMEM((chunk, d_model), jnp.float32)] * nbuf
          + [pltpu.SemaphoreType.DMA] * (2 * nbuf),
    )


def _lin1_body(n, x_ref, w_ref, deg_ref, g_ref, dis_ref):
    deg = deg_ref[0, :] + deg_ref[1, :] + 1.0
    dis = jnp.where(deg > 0, lax.rsqrt(deg), 0.0)
    h = jnp.dot(x_ref[...], w_ref[...], preferred_element_type=jnp.float32)
    g_ref[pl.ds(0, n), :] = h * dis[:n, None]
    g_ref[pl.ds(n, N_PAD - n), :] = jnp.zeros((N_PAD - n, h.shape[1]),
                                              jnp.float32)
    dis_ref[...] = dis[:, None]


def _lin2_body(a0_ref, a1_ref, g1_ref, dis_ref, w_ref, b_ref, g2_ref):
    dis = dis_ref[...]
    agg = a0_ref[...] + a1_ref[...] + g1_ref[...]
    z = jnp.maximum(agg * dis + b_ref[...], 0.0)
    h2 = jnp.dot(z, w_ref[...], preferred_element_type=jnp.float32)
    g2_ref[...] = h2 * dis


def _out_body(n, a0_ref, a1_ref, g2_ref, dis_ref, b_ref, o_ref):
    agg = a0_ref[...] + a1_ref[...] + g2_ref[...]
    o_ref[...] = (agg * dis_ref[...] + b_ref[...])[:n]


def _lin1(x, w1, deg2):
    n, d_in = x.shape
    d_hid = w1.shape[1]
    return pl.pallas_call(
        functools.partial(_lin1_body, n),
        out_shape=[
            jax.ShapeDtypeStruct((N_PAD, d_hid), jnp.float32),
            jax.ShapeDtypeStruct((N_PAD, 1), jnp.float32),
        ],
    )(x, w1, deg2)


def _lin2(a0, a1, g1, dis, w2, b1):
    d_hid, d_out = w2.shape
    grid = (N_PAD // STRIPE,)
    return pl.pallas_call(
        _lin2_body,
        grid=grid,
        in_specs=[
            pl.BlockSpec((STRIPE, d_hid), lambda i: (i, 0)),
            pl.BlockSpec((STRIPE, d_hid), lambda i: (i, 0)),
            pl.BlockSpec((STRIPE, d_hid), lambda i: (i, 0)),
            pl.BlockSpec((STRIPE, 1), lambda i: (i, 0)),
            pl.BlockSpec((d_hid, d_out), lambda i: (0, 0)),
            pl.BlockSpec((1, d_hid), lambda i: (0, 0)),
        ],
        out_specs=pl.BlockSpec((STRIPE, d_out), lambda i: (i, 0)),
        out_shape=jax.ShapeDtypeStruct((N_PAD, d_out), jnp.float32),
    )(a0, a1, g1, dis, w2, b1)


def _outk(n, a0, a1, g2, dis, b2):
    d_out = g2.shape[1]
    return pl.pallas_call(
        functools.partial(_out_body, n),
        out_shape=jax.ShapeDtypeStruct((n, d_out), jnp.float32),
    )(a0, a1, g2, dis, b2)


@jax.jit
def kernel(x, edge_index, W1, b1, W2, b2):
    n, d_in = x.shape
    d_hid = W1.shape[1]
    d_out = W2.shape[1]
    n_edges = edge_index.shape[1]
    pad = n + jnp.arange(E_PAD - n_edges, dtype=jnp.int32) % (N_PAD - n)
    ep = jnp.concatenate(
        [edge_index.astype(jnp.int32),
         jnp.broadcast_to(pad, (2, E_PAD - n_edges))], axis=1)
    src1 = ep[0]
    dst1 = ep[1]

    zeros1 = jnp.zeros((N_PAD,), jnp.float32)
    zeros_h = jnp.zeros((N_PAD, d_hid), jnp.float32)
    zeros_o = jnp.zeros((N_PAD, d_out), jnp.float32)

    deg2 = _make_deg_kernel()(dst1.reshape(E_PAD // CHUNK, CHUNK), zeros1)
    g1, dis = _lin1(x, W1, deg2)
    c1 = 64
    a10, a11 = _make_agg_kernel(d_hid, c1, 4, 2)(
        g1, src1.reshape(E_PAD // c1, c1), dst1.reshape(E_PAD // c1, c1),
        zeros_h)
    g2 = _lin2(a10, a11, g1, dis, W2, b1.reshape(1, d_hid))
    c2 = 128
    a20, a21 = _make_agg_kernel(d_out, c2, 4, 1)(
        g2, src1.reshape(E_PAD // c2, c2), dst1.reshape(E_PAD // c2, c2),
        zeros_o)
    return _outk(n, a20, a21, g2, dis, b2.reshape(1, d_out))


# speedup vs baseline: 3.4402x; 1.0321x over previous
import functools

import numpy as np

import jax
import jax.numpy as jnp
from jax import lax
from jax.experimental import pallas as pl
from jax.experimental.pallas import tpu as pltpu
from jax.experimental.pallas import tpu_sc as plsc

N_PAD = 10240
PAD_NODE = N_PAD - 8
NC = 2
NS = 16
NW = NC * NS
CHUNK = 128
CPT = 80
EPT = CPT * CHUNK
E_PAD = NW * EPT
NBUF = 4
STRIPE = N_PAD // NS


def _sc_mesh():
    return plsc.VectorSubcoreMesh(core_axis_name="c", subcore_axis_name="s")


def _deg_body(dst2d_hbm, out_hbm, dst_v, ones_v, zero_v, deg_sh, sem):
    cid = lax.axis_index("c")
    sid = lax.axis_index("s")
    gid = cid * NS + sid

    stripe0 = pl.multiple_of(sid * STRIPE, 8)
    for j in range(STRIPE // 16):
        zero_v[pl.ds(j * 16, 16)] = jnp.zeros((16,), jnp.float32)
    pltpu.sync_copy(zero_v, deg_sh.at[pl.ds(stripe0, STRIPE)])
    for j in range(CHUNK // 16):
        ones_v[pl.ds(j * 16, 16)] = jnp.ones((16,), jnp.float32)
    row0 = pl.multiple_of(gid * CPT, 8)
    pltpu.sync_copy(dst2d_hbm.at[pl.ds(row0, CPT)], dst_v)
    plsc.subcore_barrier()

    wave = 8

    def body(w, carry):
        for b in range(wave):
            pltpu.async_copy(ones_v, deg_sh.at[dst_v.at[w * wave + b]], sem,
                             add=True)
        for b in range(wave):
            pltpu.make_async_copy(ones_v, deg_sh.at[dst_v.at[0]], sem).wait()
        return carry

    lax.fori_loop(0, CPT // wave, body, 0)
    plsc.subcore_barrier()
    pltpu.sync_copy(deg_sh.at[pl.ds(stripe0, STRIPE)],
                    out_hbm.at[cid, pl.ds(stripe0, STRIPE)])


def _make_deg_kernel():
    return pl.kernel(
        _deg_body,
        out_type=jax.ShapeDtypeStruct((NC, N_PAD), jnp.float32),
        mesh=_sc_mesh(),
        compiler_params=pltpu.CompilerParams(use_tc_tiling_on_sc=False),
        scratch_types=[
            pltpu.VMEM((CPT, CHUNK), jnp.int32),
            pltpu.VMEM((CHUNK,), jnp.float32),
            pltpu.VMEM((STRIPE,), jnp.float32),
            pltpu.VMEM_SHARED((N_PAD,), jnp.float32),
            pltpu.SemaphoreType.DMA,
        ],
    )


def _agg_body(chunk, nbuf, halves, g_hbm, src2d_hbm, dst2d_hbm,
              out0_hbm, out1_hbm, src_v, dst_v, zero_v, agg_sh, *bufs):
    rows = bufs[:nbuf]
    gsem = bufs[nbuf:2 * nbuf]
    ssem = bufs[2 * nbuf:3 * nbuf]
    cpt = EPT // chunk
    cph = cpt // halves
    d_model = zero_v.shape[1]
    cid = lax.axis_index("c")
    sid = lax.axis_index("s")
    gid = cid * NS + sid

    stripe0 = pl.multiple_of(sid * STRIPE, 8)
    for r in range(zero_v.shape[0]):
        for j in range(d_model // 16):
            zero_v[r, pl.ds(j * 16, 16)] = jnp.zeros((16,), jnp.float32)

    def zbody(k, carry):
        pltpu.sync_copy(
            zero_v, agg_sh.at[pl.ds(stripe0 + k * zero_v.shape[0],
                                    zero_v.shape[0])])
        return carry

    lax.fori_loop(0, STRIPE // zero_v.shape[0], zbody, 0)
    plsc.subcore_barrier()

    for h in range(halves):
        row0 = pl.multiple_of(gid * cpt + h * cph, 8)
        pltpu.sync_copy(src2d_hbm.at[pl.ds(row0, cph)], src_v)
        pltpu.sync_copy(dst2d_hbm.at[pl.ds(row0, cph)], dst_v)

        for b in range(nbuf):
            pltpu.async_copy(g_hbm.at[src_v.at[b]], rows[b], gsem[b])

        def body(j0, carry):
            for b in range(nbuf):
                j = j0 * nbuf + b
                bp = (b - 1) % nbuf

                def drain_and_regather():
                    pltpu.make_async_copy(rows[bp], agg_sh.at[dst_v.at[0]],
                                          ssem[bp]).wait()
                    jn = j - 1 + nbuf

                    @pl.when(jn < cph)
                    def _():
                        pltpu.async_copy(g_hbm.at[src_v.at[jn]], rows[bp],
                                         gsem[bp])

                if b == 0:
                    pl.when(j0 >= 1)(drain_and_regather)
                else:
                    drain_and_regather()

                pltpu.make_async_copy(g_hbm.at[src_v.at[0]], rows[b],
                                      gsem[b]).wait()
                pltpu.async_copy(rows[b], agg_sh.at[dst_v.at[j]], ssem[b],
                                 add=True)

            return carry

        lax.fori_loop(0, cph // nbuf, body, 0)
        pltpu.make_async_copy(rows[nbuf - 1], agg_sh.at[dst_v.at[0]],
                              ssem[nbuf - 1]).wait()

    plsc.subcore_barrier()

    @pl.when(cid == 0)
    def _():
        pltpu.sync_copy(agg_sh.at[pl.ds(stripe0, STRIPE)],
                        out0_hbm.at[pl.ds(stripe0, STRIPE)])

    @pl.when(cid == 1)
    def _():
        pltpu.sync_copy(agg_sh.at[pl.ds(stripe0, STRIPE)],
                        out1_hbm.at[pl.ds(stripe0, STRIPE)])


def _make_agg_kernel(d_model, chunk, nbuf, halves):
    cph = (EPT // chunk) // halves
    return pl.kernel(
        functools.partial(_agg_body, chunk, nbuf, halves),
        out_type=[jax.ShapeDtypeStruct((N_PAD, d_model), jnp.float32),
                  jax.ShapeDtypeStruct((N_PAD, d_model), jnp.float32)],
        mesh=_sc_mesh(),
        compiler_params=pltpu.CompilerParams(use_tc_tiling_on_sc=False),
        scratch_types=[
            pltpu.VMEM((cph, chunk), jnp.int32),
            pltpu.VMEM((cph, chunk), jnp.int32),
            pltpu.VMEM((32, d_model), jnp.float32),
            pltpu.VMEM_SHARED((N_PAD, d_model), jnp.float32),
        ] + [pltpu.VMEM((chunk, d_model), jnp.float32)] * nbuf
          + [pltpu.SemaphoreType.DMA] * (2 * nbuf),
    )


def _lin1_body(n, x_ref, w_ref, deg_ref, g_ref, dis_ref):
    deg = deg_ref[0, :] + deg_ref[1, :] + 1.0
    dis = jnp.where(deg > 0, lax.rsqrt(deg), 0.0)
    h = jnp.dot(x_ref[...], w_ref[...], preferred_element_type=jnp.float32)
    g_ref[pl.ds(0, n), :] = h * dis[:n, None]
    g_ref[pl.ds(n, N_PAD - n), :] = jnp.zeros((N_PAD - n, h.shape[1]),
                                              jnp.float32)
    dis_ref[...] = dis[:, None]


def _lin2_body(a0_ref, a1_ref, g1_ref, dis_ref, w_ref, b_ref, g2_ref):
    dis = dis_ref[...]
    agg = a0_ref[...] + a1_ref[...] + g1_ref[...]
    z = jnp.maximum(agg * dis + b_ref[...], 0.0)
    h2 = jnp.dot(z, w_ref[...], preferred_element_type=jnp.float32)
    g2_ref[...] = h2 * dis


def _out_body(n, a0_ref, a1_ref, g2_ref, dis_ref, b_ref, o_ref):
    agg = a0_ref[...] + a1_ref[...] + g2_ref[...]
    o_ref[...] = (agg * dis_ref[...] + b_ref[...])[:n]


def _lin1(x, w1, deg2):
    n, d_in = x.shape
    d_hid = w1.shape[1]
    return pl.pallas_call(
        functools.partial(_lin1_body, n),
        out_shape=[
            jax.ShapeDtypeStruct((N_PAD, d_hid), jnp.float32),
            jax.ShapeDtypeStruct((N_PAD, 1), jnp.float32),
        ],
    )(x, w1, deg2)


def _lin2(a0, a1, g1, dis, w2, b1):
    d_hid, d_out = w2.shape
    grid = (N_PAD // STRIPE,)
    return pl.pallas_call(
        _lin2_body,
        grid=grid,
        in_specs=[
            pl.BlockSpec((STRIPE, d_hid), lambda i: (i, 0)),
            pl.BlockSpec((STRIPE, d_hid), lambda i: (i, 0)),
            pl.BlockSpec((STRIPE, d_hid), lambda i: (i, 0)),
            pl.BlockSpec((STRIPE, 1), lambda i: (i, 0)),
            pl.BlockSpec((d_hid, d_out), lambda i: (0, 0)),
            pl.BlockSpec((1, d_hid), lambda i: (0, 0)),
        ],
        out_specs=pl.BlockSpec((STRIPE, d_out), lambda i: (i, 0)),
        out_shape=jax.ShapeDtypeStruct((N_PAD, d_out), jnp.float32),
    )(a0, a1, g1, dis, w2, b1)


def _outk(n, a0, a1, g2, dis, b2):
    d_out = g2.shape[1]
    return pl.pallas_call(
        functools.partial(_out_body, n),
        out_shape=jax.ShapeDtypeStruct((n, d_out), jnp.float32),
    )(a0, a1, g2, dis, b2)


@jax.jit
def kernel(x, edge_index, W1, b1, W2, b2):
    n, d_in = x.shape
    d_hid = W1.shape[1]
    d_out = W2.shape[1]
    n_edges = edge_index.shape[1]
    pad = np.broadcast_to(
        (n + np.arange(E_PAD - n_edges) % (N_PAD - n)).astype(np.int32),
        (2, E_PAD - n_edges))
    ep = jnp.concatenate([edge_index.astype(jnp.int32), jnp.asarray(pad)],
                         axis=1)
    src1 = ep[0]
    dst1 = ep[1]

    deg2 = _make_deg_kernel()(dst1.reshape(E_PAD // CHUNK, CHUNK))
    g1, dis = _lin1(x, W1, deg2)
    c1 = 64
    a10, a11 = _make_agg_kernel(d_hid, c1, 4, 2)(
        g1, src1.reshape(E_PAD // c1, c1), dst1.reshape(E_PAD // c1, c1))
    g2 = _lin2(a10, a11, g1, dis, W2, b1.reshape(1, d_hid))
    c2 = 128
    a20, a21 = _make_agg_kernel(d_out, c2, 4, 1)(
        g2, src1.reshape(E_PAD // c2, c2), dst1.reshape(E_PAD // c2, c2))
    return _outk(n, a20, a21, g2, dis, b2.reshape(1, d_out))
